# baseline (speedup 1.0000x reference)
import functools

import jax
import jax.numpy as jnp
from jax import lax
from jax.experimental import pallas as pl
from jax.experimental.pallas import tpu as pltpu
from jax.experimental.pallas import tpu_sc as plsc

NC = 2
NS = 16
NW = NC * NS
L = 16

SUBR = 8
EPS = 1e-5


def _rsqrt(x):
    i = plsc.bitcast(x, jnp.int32)
    i = jnp.int32(0x5F3759DF) - (i >> 1)
    y = plsc.bitcast(i, jnp.float32)
    half = jnp.float32(0.5) * x
    for _ in range(3):
        y = y * (jnp.float32(1.5) - half * y * y)
    return y


def _make_sc_kernel(B, H, V, D):
    BR = B // NW
    J = BR // SUBR
    CHR = SUBR * H
    GPC = CHR // L
    mesh = plsc.VectorSubcoreMesh(core_axis_name="c", subcore_axis_name="s")

    @functools.partial(
        pl.kernel,
        out_type=jax.ShapeDtypeStruct((B, H, D), jnp.float32),
        mesh=mesh,
        compiler_params=pltpu.CompilerParams(
            use_tc_tiling_on_sc=False, needs_layout_passes=False
        ),
        scratch_types=[
            pltpu.VMEM((BR, H), jnp.int32),
            pltpu.VMEM((CHR, D), jnp.float32),
            pltpu.VMEM((CHR, D), jnp.float32),
            pltpu.VMEM((CHR, D), jnp.float32),
            pltpu.VMEM((CHR, D), jnp.float32),
            pltpu.VMEM((D, L), jnp.float32),
            pltpu.VMEM((D, L), jnp.float32),
            pltpu.SemaphoreType.DMA,
            pltpu.SemaphoreType.DMA,
            pltpu.SemaphoreType.DMA,
            pltpu.SemaphoreType.DMA,
        ],
    )
    def sc_embed_ln(idx_hbm, table_hbm, gb_hbm, bb_hbm, out_hbm,
                    idx_v, in0, in1, out0, out1, gbv, bbv,
                    is0, is1, os0, os1):
        wid = lax.axis_index("c") * NS + lax.axis_index("s")
        pltpu.sync_copy(idx_hbm.at[pl.ds(wid * BR, BR)], idx_v)
        pltpu.sync_copy(gb_hbm, gbv)
        pltpu.sync_copy(bb_hbm, bbv)

        ins = (in0, in1)
        outs = (out0, out1)
        isems = (is0, is1)
        osems = (os0, os1)

        def issue_gather(j, b):
            for k in range(SUBR):
                pltpu.async_copy(
                    table_hbm.at[idx_v.at[j * SUBR + k]],
                    ins[b].at[pl.ds(k * H, H)],
                    isems[b],
                )

        def wait_gather(b):
            for k in range(SUBR):
                pltpu.make_async_copy(
                    table_hbm.at[pl.ds(0, H)],
                    ins[b].at[pl.ds(k * H, H)],
                    isems[b],
                ).wait()

        def issue_out(j, b):
            for k in range(SUBR):
                pltpu.async_copy(
                    outs[b].at[pl.ds(k * H, H)],
                    out_hbm.at[wid * BR + j * SUBR + k],
                    osems[b],
                )

        def wait_out(b):
            for k in range(SUBR):
                pltpu.make_async_copy(
                    outs[b].at[pl.ds(k * H, H)],
                    out_hbm.at[0],
                    osems[b],
                ).wait()

        inv_d = jnp.float32(1.0 / D)

        NACC = 4

        def compute(b):
            in_ref = ins[b]
            out_ref = outs[b]
            lanes = lax.iota(jnp.int32, L)

            @pl.loop(0, GPC, unroll=2)
            def _group(g):
                r = lanes + g * L
                v = []
                s = [None] * NACC
                q = [None] * NACC
                for d in range(D):
                    col = jnp.full((L,), d, dtype=jnp.int32)
                    x = plsc.load_gather(in_ref, [r, col])
                    v.append(x)
                    a = d % NACC
                    s[a] = x if s[a] is None else s[a] + x
                    q[a] = x * x if q[a] is None else q[a] + x * x
                st = (s[0] + s[1]) + (s[2] + s[3])
                qt = (q[0] + q[1]) + (q[2] + q[3])
                mean = st * inv_d
                var = qt * inv_d - mean * mean
                rstd = _rsqrt(var + jnp.float32(EPS))
                for d in range(D):
                    col = jnp.full((L,), d, dtype=jnp.int32)
                    y = (v[d] - mean) * rstd * gbv[d] + bbv[d]
                    plsc.store_scatter(out_ref, [r, col], y)

        issue_gather(0, 0)
        issue_gather(1, 1)
        for b in range(2):
            wait_gather(b)
            compute(b)
            issue_out(b, b)
            issue_gather(b + 2, b)

        @pl.loop(2, J - 2, step=2)
        def _main(j0):
            for b in range(2):
                j = j0 + b
                wait_gather(b)
                wait_out(b)
                compute(b)
                issue_out(j, b)
                issue_gather(j + 2, b)

        for b in range(2):
            wait_gather(b)
            wait_out(b)
            compute(b)
            issue_out(J - 2 + b, b)
        for b in range(2):
            wait_out(b)

    return sc_embed_ln


def kernel(input_ids, table, gamma, beta):
    B, H = input_ids.shape
    V, D = table.shape
    idx = input_ids.astype(jnp.int32)
    gb = jnp.broadcast_to(gamma.astype(jnp.float32)[:, None], (D, L))
    bb = jnp.broadcast_to(beta.astype(jnp.float32)[:, None], (D, L))
    return _make_sc_kernel(B, H, V, D)(idx, table, gb, bb)


# speedup vs baseline: 1.4479x; 1.4479x over previous
import functools

import jax
import jax.numpy as jnp
from jax import lax
from jax.experimental import pallas as pl
from jax.experimental.pallas import tpu as pltpu
from jax.experimental.pallas import tpu_sc as plsc

NC = 2
NS = 16
NW = NC * NS
L = 16

SUBR = 8
EPS = 1e-5


def _rsqrt(x):
    i = plsc.bitcast(x, jnp.int32)
    i = jnp.int32(0x5F3759DF) - (i >> 1)
    y = plsc.bitcast(i, jnp.float32)
    half = jnp.float32(0.5) * x
    for _ in range(3):
        y = y * (jnp.float32(1.5) - half * y * y)
    return y


def _make_sc_kernel(B, H, V, D):
    BR = B // NW
    J = BR // SUBR
    CHR = SUBR * H
    GPC = CHR // L
    mesh = plsc.VectorSubcoreMesh(core_axis_name="c", subcore_axis_name="s")

    @functools.partial(
        pl.kernel,
        out_type=jax.ShapeDtypeStruct((B, H, D), jnp.float32),
        mesh=mesh,
        compiler_params=pltpu.CompilerParams(
            use_tc_tiling_on_sc=False, needs_layout_passes=False
        ),
        scratch_types=[
            pltpu.VMEM((BR, H), jnp.int32),
            pltpu.VMEM((CHR, D), jnp.float32),
            pltpu.VMEM((CHR, D), jnp.float32),
            pltpu.VMEM((CHR, D), jnp.float32),
            pltpu.VMEM((CHR, D), jnp.float32),
            pltpu.VMEM((D, L), jnp.float32),
            pltpu.VMEM((D, L), jnp.float32),
            pltpu.SemaphoreType.DMA,
            pltpu.SemaphoreType.DMA,
            pltpu.SemaphoreType.DMA,
            pltpu.SemaphoreType.DMA,
        ],
    )
    def sc_embed_ln(idx_hbm, table_hbm, gb_hbm, bb_hbm, out_hbm,
                    idx_v, in0, in1, out0, out1, gbv, bbv,
                    is0, is1, os0, os1):
        wid = lax.axis_index("c") * NS + lax.axis_index("s")
        pltpu.sync_copy(idx_hbm.at[pl.ds(wid * BR, BR)], idx_v)
        pltpu.sync_copy(gb_hbm, gbv)
        pltpu.sync_copy(bb_hbm, bbv)

        ins = (in0, in1)
        outs = (out0, out1)
        isems = (is0, is1)
        osems = (os0, os1)

        def issue_gather(j, b):
            for k in range(SUBR):
                pltpu.async_copy(
                    table_hbm.at[idx_v.at[j * SUBR + k]],
                    ins[b].at[pl.ds(k * H, H)],
                    isems[b],
                )

        def wait_gather(b):
            for k in range(SUBR):
                pltpu.make_async_copy(
                    table_hbm.at[pl.ds(0, H)],
                    ins[b].at[pl.ds(k * H, H)],
                    isems[b],
                ).wait()

        def issue_out(j, b):
            for k in range(SUBR):
                pltpu.async_copy(
                    outs[b].at[pl.ds(k * H, H)],
                    out_hbm.at[wid * BR + j * SUBR + k],
                    osems[b],
                )

        def wait_out(b):
            for k in range(SUBR):
                pltpu.make_async_copy(
                    outs[b].at[pl.ds(k * H, H)],
                    out_hbm.at[0],
                    osems[b],
                ).wait()

        inv_d = jnp.float32(1.0 / D)

        NACC = 4

        def compute(b):
            in_ref = ins[b]
            out_ref = outs[b]
            lanes = lax.iota(jnp.int32, L)

            @pl.loop(0, GPC, unroll=2)
            def _group(g):
                r = lanes + g * L
                v = []
                s = [None] * NACC
                q = [None] * NACC
                for d in range(D):
                    col = (lanes + d) & (D - 1)
                    x = plsc.load_gather(in_ref, [r, col])
                    v.append(x)
                    a = d % NACC
                    s[a] = x if s[a] is None else s[a] + x
                    q[a] = x * x if q[a] is None else q[a] + x * x
                st = (s[0] + s[1]) + (s[2] + s[3])
                qt = (q[0] + q[1]) + (q[2] + q[3])
                mean = st * inv_d
                var = qt * inv_d - mean * mean
                rstd = _rsqrt(var + jnp.float32(EPS))
                for d in range(D):
                    col = (lanes + d) & (D - 1)
                    y = (v[d] - mean) * rstd * gbv[d] + bbv[d]
                    plsc.store_scatter(out_ref, [r, col], y)

        issue_gather(0, 0)
        issue_gather(1, 1)
        for b in range(2):
            wait_gather(b)
            compute(b)
            issue_out(b, b)
            issue_gather(b + 2, b)

        @pl.loop(2, J - 2, step=2)
        def _main(j0):
            for b in range(2):
                j = j0 + b
                wait_gather(b)
                wait_out(b)
                compute(b)
                issue_out(j, b)
                issue_gather(j + 2, b)

        for b in range(2):
            wait_gather(b)
            wait_out(b)
            compute(b)
            issue_out(J - 2 + b, b)
        for b in range(2):
            wait_out(b)

    return sc_embed_ln


def kernel(input_ids, table, gamma, beta):
    B, H = input_ids.shape
    V, D = table.shape
    idx = input_ids.astype(jnp.int32)
    rot = (jnp.arange(D)[:, None] + jnp.arange(L)[None, :]) % D
    gb = gamma.astype(jnp.float32)[rot]
    bb = beta.astype(jnp.float32)[rot]
    return _make_sc_kernel(B, H, V, D)(idx, table, gb, bb)


# speedup vs baseline: 2.1268x; 1.4688x over previous
import functools

import jax
import jax.numpy as jnp
from jax import lax
from jax.experimental import pallas as pl
from jax.experimental.pallas import tpu as pltpu
from jax.experimental.pallas import tpu_sc as plsc

NC = 2
NS = 16
NW = NC * NS
L = 16
BLK = 128
EPS = 1e-5


def _rsqrt(x):
    i = plsc.bitcast(x, jnp.int32)
    i = jnp.int32(0x5F3759DF) - (i >> 1)
    y = plsc.bitcast(i, jnp.float32)
    half = jnp.float32(0.5) * x
    for _ in range(3):
        y = y * (jnp.float32(1.5) - half * y * y)
    return y


def _make_sc_kernel(B, H, V, D):
    BR = B // NW
    NBLK = BR // BLK
    GPB = BLK // L
    D8 = D // 8
    mesh = plsc.VectorSubcoreMesh(core_axis_name="c", subcore_axis_name="s")

    @functools.partial(
        pl.kernel,
        out_type=jax.ShapeDtypeStruct((H, D8, B // BLK, 8, BLK), jnp.float32),
        mesh=mesh,
        compiler_params=pltpu.CompilerParams(
            use_tc_tiling_on_sc=False, needs_layout_passes=False
        ),
        scratch_types=[
            pltpu.VMEM((BR, H), jnp.int32),
            pltpu.VMEM((H, BR), jnp.int32),
            pltpu.VMEM((BLK, D), jnp.float32),
            pltpu.VMEM((BLK, D), jnp.float32),
            pltpu.VMEM((D, BLK), jnp.float32),
            pltpu.VMEM((D, BLK), jnp.float32),
            pltpu.VMEM((D, L), jnp.float32),
            pltpu.VMEM((D, L), jnp.float32),
            pltpu.SemaphoreType.DMA,
            pltpu.SemaphoreType.DMA,
            pltpu.SemaphoreType.DMA,
            pltpu.SemaphoreType.DMA,
        ],
    )
    def sc_embed_ln(idx_hbm, table_hbm, gb_hbm, bb_hbm, out_hbm,
                    idx_v, idx_t, in0, in1, out0, out1, gbv, bbv,
                    is0, is1, os0, os1):
        wid = lax.axis_index("c") * NS + lax.axis_index("s")
        pltpu.sync_copy(idx_hbm.at[pl.ds(wid * BR, BR)], idx_v)
        pltpu.sync_copy(gb_hbm, gbv)
        pltpu.sync_copy(bb_hbm, bbv)

        lanes = lax.iota(jnp.int32, L)

        @pl.loop(0, H)
        def _trans(h):
            hcol = jnp.zeros((L,), dtype=jnp.int32) + h
            for c in range(BR // L):
                rr = lanes + c * L
                vals = plsc.load_gather(idx_v, [rr, hcol])
                plsc.store_scatter(idx_t, [hcol, rr], vals)

        ins = (in0, in1)
        outs = (out0, out1)
        isems = (is0, is1)
        osems = (os0, os1)

        def issue_gather(blk, h, b):
            pltpu.async_copy(
                table_hbm.at[idx_t.at[h, pl.ds(blk * BLK, BLK)]],
                ins[b],
                isems[b],
            )

        def wait_gather(b):
            pltpu.make_async_copy(
                table_hbm.at[pl.ds(0, BLK)],
                ins[b],
                isems[b],
            ).wait()

        def issue_out(gblk, h, b):
            for d8 in range(D8):
                pltpu.async_copy(
                    outs[b].at[pl.ds(d8 * 8, 8)],
                    out_hbm.at[h, d8, gblk],
                    osems[b],
                )

        def wait_out(b):
            for d8 in range(D8):
                pltpu.make_async_copy(
                    outs[b].at[pl.ds(d8 * 8, 8)],
                    out_hbm.at[0, 0, 0],
                    osems[b],
                ).wait()

        inv_d = jnp.float32(1.0 / D)
        NACC = 4

        def compute(b):
            in_ref = ins[b]
            out_ref = outs[b]

            @pl.loop(0, GPB, unroll=2)
            def _group(g):
                r = lanes + g * L
                v = []
                s = [None] * NACC
                q = [None] * NACC
                for d in range(D):
                    col = (lanes + d) & (D - 1)
                    x = plsc.load_gather(in_ref, [r, col])
                    v.append(x)
                    a = d % NACC
                    s[a] = x if s[a] is None else s[a] + x
                    q[a] = x * x if q[a] is None else q[a] + x * x
                st = (s[0] + s[1]) + (s[2] + s[3])
                qt = (q[0] + q[1]) + (q[2] + q[3])
                mean = st * inv_d
                var = qt * inv_d - mean * mean
                rstd = _rsqrt(var + jnp.float32(EPS))
                for d in range(D):
                    col = (lanes + d) & (D - 1)
                    y = (v[d] - mean) * rstd * gbv[d] + bbv[d]
                    plsc.store_scatter(out_ref, [col, r], y)

        @pl.loop(0, NBLK)
        def _blk(blk):
            gblk = wid * NBLK + blk
            issue_gather(blk, 0, 0)
            issue_gather(blk, 1, 1)
            for h in range(2):
                wait_gather(h)
                compute(h)
                issue_out(gblk, h, h)
                issue_gather(blk, h + 2, h)

            @pl.loop(2, H - 2, step=2)
            def _main(h0):
                for b in range(2):
                    h = h0 + b
                    wait_gather(b)
                    wait_out(b)
                    compute(b)
                    issue_out(gblk, h, b)
                    issue_gather(blk, h + 2, b)

            for b in range(2):
                wait_gather(b)
                wait_out(b)
                compute(b)
                issue_out(gblk, H - 2 + b, b)
            for b in range(2):
                wait_out(b)

    return sc_embed_ln


def kernel(input_ids, table, gamma, beta):
    B, H = input_ids.shape
    V, D = table.shape
    idx = input_ids.astype(jnp.int32)
    rot = (jnp.arange(D)[:, None] + jnp.arange(L)[None, :]) % D
    gb = gamma.astype(jnp.float32)[rot]
    bb = beta.astype(jnp.float32)[rot]
    out5 = _make_sc_kernel(B, H, V, D)(idx, table, gb, bb)
    return out5.transpose(2, 4, 0, 1, 3).reshape(B, H, D)
